# baseline (speedup 1.0000x reference)
import functools

import jax
import jax.numpy as jnp
from jax import lax
from jax.experimental import pallas as pl
from jax.experimental.pallas import tpu as pltpu

D_MODEL = 768
N_HEAD = 12
D_HEAD = D_MODEL // N_HEAD
SEQ = 2048
_SCALE = 1.0 / (D_HEAD ** 0.5)
_CH = 128
_NCH = SEQ // _CH


def _dot_t(a, b):
    return lax.dot_general(a, b, (((1,), (1,)), ((), ())),
                           preferred_element_type=jnp.float32)


def _dot(a, b):
    return lax.dot_general(a, b, (((1,), (0,)), ((), ())),
                           preferred_element_type=jnp.float32)


def _proj_kernel(kv_ref, wkv_ref, k_ref, v1_ref, kv1_ref):
    kvp = _dot_t(kv_ref[...], wkv_ref[...])
    kk = kvp[:, :D_HEAD]
    vv = kvp[:, D_HEAD:]
    kn = jnp.sqrt(jnp.sum(kk * kk, axis=0, keepdims=True))
    vn = jnp.sqrt(jnp.sum(vv * vv, axis=0, keepdims=True))
    kk = kk / jnp.maximum(kn, 1e-12)
    vv = vv / jnp.maximum(vn, 1e-12)
    ones = jnp.ones((SEQ, 1), jnp.float32)
    k_ref[...] = kk
    v1_ref[...] = jnp.concatenate([vv, ones], axis=1)
    kv1_ref[...] = jnp.concatenate([kk, vv, ones], axis=1)


def _head(qh, k, v1, kv1, gate):
    s = _dot_t(qh, k)
    m = jnp.max(s, axis=1, keepdims=True)
    rkv = jnp.zeros((SEQ, 2 * D_HEAD + 1), jnp.float32)
    pv = jnp.zeros((SEQ, D_HEAD + 1), jnp.float32)
    for c in range(_NCH):
        sc = s[:, c * _CH:(c + 1) * _CH]
        ohc = (sc >= m).astype(jnp.float32)
        pc = jnp.exp(sc * _SCALE)
        rkv = rkv + _dot(ohc, kv1[c * _CH:(c + 1) * _CH, :])
        pv = pv + _dot(pc, v1[c * _CH:(c + 1) * _CH, :])
    local_out = pv[:, :D_HEAD] / pv[:, D_HEAD:]
    pr = jnp.zeros((SEQ, D_HEAD + 1), jnp.float32)
    for c in range(_NCH):
        rkvc = rkv[c * _CH:(c + 1) * _CH, :]
        s2c = _dot_t(qh, rkvc[:, :D_HEAD])
        p2c = jnp.exp(s2c * _SCALE)
        pr = pr + _dot(p2c, rkvc[:, D_HEAD:])
    r_out = pr[:, :D_HEAD] / pr[:, D_HEAD:]
    return r_out * gate + local_out * (1.0 - gate)


def _main_kernel(q_ref, k_ref, v1_ref, kv1_ref, wq_ref, wct_ref, bias_ref,
                 out_ref):
    t = pl.program_id(0)
    gate = jax.nn.sigmoid(bias_ref[...])
    k = k_ref[...]
    v1 = v1_ref[...]
    kv1 = kv1_ref[...]

    qh2 = _dot_t(q_ref[...], wq_ref[...])
    out_a = _head(qh2[:, :D_HEAD], k, v1, kv1, gate)
    out_b = _head(qh2[:, D_HEAD:], k, v1, kv1, gate)
    contrib = _dot(jnp.concatenate([out_a, out_b], axis=1), wct_ref[...])

    @pl.when(t == 0)
    def _init():
        out_ref[...] = contrib

    @pl.when(t != 0)
    def _acc():
        out_ref[...] += contrib


@functools.partial(jax.jit, static_argnames=())
def kernel(q, kv, w_q, w_kv, w_concat, bias):
    b, l, dm = q.shape
    q2 = q.reshape(l, dm)
    kv2 = kv.reshape(l, dm)
    wct = w_concat.T
    bias2 = bias.reshape(1, D_HEAD)

    k_n, v1_n, kv1_n = pl.pallas_call(
        _proj_kernel,
        out_shape=[
            jax.ShapeDtypeStruct((l, D_HEAD), jnp.float32),
            jax.ShapeDtypeStruct((l, D_HEAD + 1), jnp.float32),
            jax.ShapeDtypeStruct((l, 2 * D_HEAD + 1), jnp.float32),
        ],
    )(kv2, w_kv)

    out = pl.pallas_call(
        _main_kernel,
        grid=(N_HEAD // 2,),
        in_specs=[
            pl.BlockSpec((l, dm), lambda t: (0, 0)),
            pl.BlockSpec((l, D_HEAD), lambda t: (0, 0)),
            pl.BlockSpec((l, D_HEAD + 1), lambda t: (0, 0)),
            pl.BlockSpec((l, 2 * D_HEAD + 1), lambda t: (0, 0)),
            pl.BlockSpec((2 * D_HEAD, dm), lambda t: (t, 0)),
            pl.BlockSpec((2 * D_HEAD, dm), lambda t: (t, 0)),
            pl.BlockSpec((1, D_HEAD), lambda t: (0, 0)),
        ],
        out_specs=pl.BlockSpec((l, dm), lambda t: (0, 0)),
        out_shape=jax.ShapeDtypeStruct((l, dm), jnp.float32),
        compiler_params=pltpu.CompilerParams(
            dimension_semantics=("arbitrary",),
        ),
    )(q2, k_n, v1_n, kv1_n, w_q, wct, bias2)
    return out.reshape(b, l, dm)


# speedup vs baseline: 1.1900x; 1.1900x over previous
import functools

import jax
import jax.numpy as jnp
from jax import lax
from jax.experimental import pallas as pl
from jax.experimental.pallas import tpu as pltpu
from jax.experimental.pallas import tpu_sc as plsc

D_MODEL = 768
N_HEAD = 12
D_HEAD = D_MODEL // N_HEAD
SEQ = 2048
_SCALE = 1.0 / (D_HEAD ** 0.5)
_CH = 256
_NCH = SEQ // _CH
_NPAIR = N_HEAD // 2

_SC_NC = 2
_SC_NS = 16
_SC_NW = _SC_NC * _SC_NS
_B_TOT = N_HEAD * SEQ
_B_PER_W = _B_TOT // _SC_NW


def _dot_t(a, b):
    return lax.dot_general(a, b, (((1,), (1,)), ((), ())),
                           preferred_element_type=jnp.float32)


def _dot(a, b):
    return lax.dot_general(a, b, (((1,), (0,)), ((), ())),
                           preferred_element_type=jnp.float32)


def _proj_kernel(kv_ref, wkv_ref, k_ref, v1_ref, kvn_ref):
    kvp = _dot_t(kv_ref[...], wkv_ref[...])
    kk = kvp[:, :D_HEAD]
    vv = kvp[:, D_HEAD:]
    kn = jnp.sqrt(jnp.sum(kk * kk, axis=0, keepdims=True))
    vn = jnp.sqrt(jnp.sum(vv * vv, axis=0, keepdims=True))
    kk = kk / jnp.maximum(kn, 1e-12)
    vv = vv / jnp.maximum(vn, 1e-12)
    ones = jnp.ones((SEQ, 1), jnp.float32)
    k_ref[...] = kk
    v1_ref[...] = jnp.concatenate([vv, ones], axis=1)
    kvn_ref[...] = jnp.concatenate([kk, vv], axis=1)


def _half_a(qh, k, v1):
    s = _dot_t(qh, k)
    m = jnp.max(s, axis=1, keepdims=True)
    idxm = jnp.full((SEQ, 1), SEQ, jnp.int32)
    pv = jnp.zeros((SEQ, D_HEAD + 1), jnp.float32)
    for c in range(_NCH):
        sc = s[:, c * _CH:(c + 1) * _CH]
        col = lax.broadcasted_iota(jnp.int32, (SEQ, _CH), 1) + c * _CH
        idxc = jnp.min(jnp.where(sc >= m, col, SEQ), axis=1, keepdims=True)
        idxm = jnp.minimum(idxm, idxc)
        pc = jnp.exp(sc * _SCALE)
        pv = pv + _dot(pc, v1[c * _CH:(c + 1) * _CH, :])
    local_out = pv[:, :D_HEAD] / pv[:, D_HEAD:]
    return idxm, local_out


def _stage_a(q_ref, k_ref, v1_ref, wq_ref, wct_ref, bias_ref,
             part_ref, qh_ref, idx_ref):
    t = pl.program_id(0)
    gate = jax.nn.sigmoid(bias_ref[...])
    k = k_ref[...]
    v1 = v1_ref[...]
    qh2 = _dot_t(q_ref[...], wq_ref[...])
    idx_a, lo_a = _half_a(qh2[:, :D_HEAD], k, v1)
    idx_b, lo_b = _half_a(qh2[:, D_HEAD:], k, v1)
    gate2 = jnp.concatenate([gate, gate], axis=1)
    contrib = _dot(jnp.concatenate([lo_a, lo_b], axis=1) * (1.0 - gate2),
                   wct_ref[...])

    @pl.when(t == 0)
    def _init():
        part_ref[...] = contrib

    @pl.when(t != 0)
    def _acc():
        part_ref[...] += contrib

    qh_ref[0] = qh2
    idx_ref[0] = jnp.concatenate([idx_a, idx_b], axis=1)


def _half_b(qh, rkv_ref, half):
    pr = jnp.zeros((SEQ, D_HEAD + 1), jnp.float32)
    ones = jnp.ones((_CH, 1), jnp.float32)
    for c in range(_NCH):
        rkvc = rkv_ref[0, c * _CH:(c + 1) * _CH, half, :]
        s2c = _dot_t(qh, rkvc[:, :D_HEAD])
        p2c = jnp.exp(s2c * _SCALE)
        pr = pr + _dot(p2c, jnp.concatenate([rkvc[:, D_HEAD:], ones], axis=1))
    return pr[:, :D_HEAD] / pr[:, D_HEAD:]


def _stage_b(qh_ref, rkv_ref, wct_ref, bias_ref, part_ref, out_ref):
    t = pl.program_id(0)
    gate = jax.nn.sigmoid(bias_ref[...])
    qh2 = qh_ref[0]
    r_a = _half_b(qh2[:, :D_HEAD], rkv_ref, 0)
    r_b = _half_b(qh2[:, D_HEAD:], rkv_ref, 1)
    gate2 = jnp.concatenate([gate, gate], axis=1)
    contrib = _dot(jnp.concatenate([r_a, r_b], axis=1) * gate2, wct_ref[...])

    @pl.when(t == 0)
    def _init():
        out_ref[...] = part_ref[...] + contrib

    @pl.when(t != 0)
    def _acc():
        out_ref[...] += contrib


def _sc_gather(table, idx_flat):
    mesh = plsc.VectorSubcoreMesh(core_axis_name="c", subcore_axis_name="s")

    @functools.partial(
        pl.kernel, mesh=mesh,
        out_type=jax.ShapeDtypeStruct((_B_TOT, 2 * D_HEAD), jnp.float32),
        scratch_types=[
            pltpu.VMEM((_B_PER_W,), jnp.int32),
            pltpu.VMEM((_B_PER_W, 2 * D_HEAD), jnp.float32),
            pltpu.SemaphoreType.DMA,
        ],
    )
    def _g(table_hbm, idx_hbm, out_hbm, idx_v, rows_v, sem):
        wid = lax.axis_index("s") * _SC_NC + lax.axis_index("c")
        base = wid * _B_PER_W
        pltpu.sync_copy(idx_hbm.at[pl.ds(base, _B_PER_W)], idx_v)
        pltpu.async_copy(table_hbm.at[idx_v], rows_v, sem).wait()
        pltpu.sync_copy(rows_v, out_hbm.at[pl.ds(base, _B_PER_W)])

    return _g(table, idx_flat)


@functools.partial(jax.jit, static_argnames=())
def kernel(q, kv, w_q, w_kv, w_concat, bias):
    b, l, dm = q.shape
    q2 = q.reshape(l, dm)
    kv2 = kv.reshape(l, dm)
    wct = w_concat.T
    bias2 = bias.reshape(1, D_HEAD)

    k_n, v1_n, kvn = pl.pallas_call(
        _proj_kernel,
        out_shape=[
            jax.ShapeDtypeStruct((l, D_HEAD), jnp.float32),
            jax.ShapeDtypeStruct((l, D_HEAD + 1), jnp.float32),
            jax.ShapeDtypeStruct((l, 2 * D_HEAD), jnp.float32),
        ],
    )(kv2, w_kv)

    part, qh_all, idx_all = pl.pallas_call(
        _stage_a,
        grid=(_NPAIR,),
        in_specs=[
            pl.BlockSpec((l, dm), lambda t: (0, 0)),
            pl.BlockSpec((l, D_HEAD), lambda t: (0, 0)),
            pl.BlockSpec((l, D_HEAD + 1), lambda t: (0, 0)),
            pl.BlockSpec((2 * D_HEAD, dm), lambda t: (t, 0)),
            pl.BlockSpec((2 * D_HEAD, dm), lambda t: (t, 0)),
            pl.BlockSpec((1, D_HEAD), lambda t: (0, 0)),
        ],
        out_specs=[
            pl.BlockSpec((l, dm), lambda t: (0, 0)),
            pl.BlockSpec((1, l, 2 * D_HEAD), lambda t: (t, 0, 0)),
            pl.BlockSpec((1, l, 2), lambda t: (t, 0, 0)),
        ],
        out_shape=[
            jax.ShapeDtypeStruct((l, dm), jnp.float32),
            jax.ShapeDtypeStruct((_NPAIR, l, 2 * D_HEAD), jnp.float32),
            jax.ShapeDtypeStruct((_NPAIR, l, 2), jnp.int32),
        ],
        compiler_params=pltpu.CompilerParams(
            dimension_semantics=("arbitrary",),
        ),
    )(q2, k_n, v1_n, w_q, wct, bias2)

    idx_flat = idx_all.reshape(_B_TOT)
    rkv_all = _sc_gather(kvn, idx_flat).reshape(_NPAIR, l, 2, 2 * D_HEAD)

    out = pl.pallas_call(
        _stage_b,
        grid=(_NPAIR,),
        in_specs=[
            pl.BlockSpec((1, l, 2 * D_HEAD), lambda t: (t, 0, 0)),
            pl.BlockSpec((1, l, 2, 2 * D_HEAD), lambda t: (t, 0, 0, 0)),
            pl.BlockSpec((2 * D_HEAD, dm), lambda t: (t, 0)),
            pl.BlockSpec((1, D_HEAD), lambda t: (0, 0)),
            pl.BlockSpec((l, dm), lambda t: (0, 0)),
        ],
        out_specs=pl.BlockSpec((l, dm), lambda t: (0, 0)),
        out_shape=jax.ShapeDtypeStruct((l, dm), jnp.float32),
        compiler_params=pltpu.CompilerParams(
            dimension_semantics=("arbitrary",),
        ),
    )(qh_all, rkv_all, wct, bias2, part)
    return out.reshape(b, l, dm)


# speedup vs baseline: 1.7867x; 1.5015x over previous
import functools

import jax
import jax.numpy as jnp
from jax import lax
from jax.experimental import pallas as pl
from jax.experimental.pallas import tpu as pltpu

D_MODEL = 768
N_HEAD = 12
D_HEAD = D_MODEL // N_HEAD
SEQ = 2048
_SCALE = 1.0 / (D_HEAD ** 0.5)
_CH = 256
_NCH = SEQ // _CH
_CH2 = 512
_NCH2 = SEQ // _CH2


def _dot_t(a, b):
    return lax.dot_general(a, b, (((1,), (1,)), ((), ())),
                           preferred_element_type=jnp.float32)


def _dot(a, b):
    return lax.dot_general(a, b, (((1,), (0,)), ((), ())),
                           preferred_element_type=jnp.float32)


def _proj_kernel(kv_ref, wkv_ref, k_ref, v1_ref, kv1_ref):
    kvp = _dot_t(kv_ref[...], wkv_ref[...])
    kk = kvp[:, :D_HEAD]
    vv = kvp[:, D_HEAD:]
    kn = jnp.sqrt(jnp.sum(kk * kk, axis=0, keepdims=True))
    vn = jnp.sqrt(jnp.sum(vv * vv, axis=0, keepdims=True))
    kk = kk / jnp.maximum(kn, 1e-12)
    vv = vv / jnp.maximum(vn, 1e-12)
    ones = jnp.ones((SEQ, 1), jnp.float32)
    k_ref[...] = kk
    v1_ref[...] = jnp.concatenate([vv, ones], axis=1)
    kv1_ref[...] = jnp.concatenate([kk, vv, ones], axis=1)


def _head(qh, k, v1, kv1, gate):
    s = _dot_t(qh, k)
    m = jnp.max(s, axis=1, keepdims=True)
    rkv = jnp.zeros((SEQ, 2 * D_HEAD + 1), jnp.float32)
    pv = jnp.zeros((SEQ, D_HEAD + 1), jnp.float32)
    for c in range(_NCH):
        sc = s[:, c * _CH:(c + 1) * _CH]
        ohc = (sc >= m).astype(jnp.float32)
        pc = jnp.exp(sc * _SCALE)
        rkv = rkv + _dot(ohc, kv1[c * _CH:(c + 1) * _CH, :])
        pv = pv + _dot(pc, v1[c * _CH:(c + 1) * _CH, :])
    local_out = pv[:, :D_HEAD] / pv[:, D_HEAD:]
    pr = jnp.zeros((SEQ, D_HEAD + 1), jnp.float32)
    for c in range(_NCH2):
        rkvc = rkv[c * _CH2:(c + 1) * _CH2, :]
        s2c = _dot_t(qh, rkvc[:, :D_HEAD])
        p2c = jnp.exp(s2c * _SCALE)
        pr = pr + _dot(p2c, rkvc[:, D_HEAD:])
    r_out = pr[:, :D_HEAD] / pr[:, D_HEAD:]
    return r_out * gate + local_out * (1.0 - gate)


def _main_kernel(q_ref, k_ref, v1_ref, kv1_ref, wq_ref, wct_ref, bias_ref,
                 out_ref):
    t = pl.program_id(0)
    gate = jax.nn.sigmoid(bias_ref[...])
    k = k_ref[...]
    v1 = v1_ref[...]
    kv1 = kv1_ref[...]

    qh2 = _dot_t(q_ref[...], wq_ref[...])
    out_a = _head(qh2[:, :D_HEAD], k, v1, kv1, gate)
    out_b = _head(qh2[:, D_HEAD:], k, v1, kv1, gate)
    contrib = _dot(jnp.concatenate([out_a, out_b], axis=1), wct_ref[...])

    @pl.when(t == 0)
    def _init():
        out_ref[...] = contrib

    @pl.when(t != 0)
    def _acc():
        out_ref[...] += contrib


@functools.partial(jax.jit, static_argnames=())
def kernel(q, kv, w_q, w_kv, w_concat, bias):
    b, l, dm = q.shape
    q2 = q.reshape(l, dm)
    kv2 = kv.reshape(l, dm)
    wct = w_concat.T
    bias2 = bias.reshape(1, D_HEAD)

    k_n, v1_n, kv1_n = pl.pallas_call(
        _proj_kernel,
        out_shape=[
            jax.ShapeDtypeStruct((l, D_HEAD), jnp.float32),
            jax.ShapeDtypeStruct((l, D_HEAD + 1), jnp.float32),
            jax.ShapeDtypeStruct((l, 2 * D_HEAD + 1), jnp.float32),
        ],
    )(kv2, w_kv)

    out = pl.pallas_call(
        _main_kernel,
        grid=(N_HEAD // 2,),
        in_specs=[
            pl.BlockSpec((l, dm), lambda t: (0, 0)),
            pl.BlockSpec((l, D_HEAD), lambda t: (0, 0)),
            pl.BlockSpec((l, D_HEAD + 1), lambda t: (0, 0)),
            pl.BlockSpec((l, 2 * D_HEAD + 1), lambda t: (0, 0)),
            pl.BlockSpec((2 * D_HEAD, dm), lambda t: (t, 0)),
            pl.BlockSpec((2 * D_HEAD, dm), lambda t: (t, 0)),
            pl.BlockSpec((1, D_HEAD), lambda t: (0, 0)),
        ],
        out_specs=pl.BlockSpec((l, dm), lambda t: (0, 0)),
        out_shape=jax.ShapeDtypeStruct((l, dm), jnp.float32),
        compiler_params=pltpu.CompilerParams(
            dimension_semantics=("arbitrary",),
        ),
    )(q2, k_n, v1_n, kv1_n, w_q, wct, bias2)
    return out.reshape(b, l, dm)


# speedup vs baseline: 1.8541x; 1.0377x over previous
import functools

import jax
import jax.numpy as jnp
from jax import lax
from jax.experimental import pallas as pl
from jax.experimental.pallas import tpu as pltpu

D_MODEL = 768
N_HEAD = 12
D_HEAD = D_MODEL // N_HEAD
SEQ = 2048
_SCALE = 1.0 / (D_HEAD ** 0.5)
_CH = 256
_NCH = SEQ // _CH


def _dot_t(a, b):
    return lax.dot_general(a, b, (((1,), (1,)), ((), ())),
                           preferred_element_type=jnp.float32)


def _dot(a, b):
    return lax.dot_general(a, b, (((1,), (0,)), ((), ())),
                           preferred_element_type=jnp.float32)


def _proj_kernel(kv_ref, wkv_ref, k_ref, v1_ref, kv1_ref):
    kvp = _dot_t(kv_ref[...], wkv_ref[...])
    kk = kvp[:, :D_HEAD]
    vv = kvp[:, D_HEAD:]
    kn = jnp.sqrt(jnp.sum(kk * kk, axis=0, keepdims=True))
    vn = jnp.sqrt(jnp.sum(vv * vv, axis=0, keepdims=True))
    kk = kk / jnp.maximum(kn, 1e-12)
    vv = vv / jnp.maximum(vn, 1e-12)
    ones = jnp.ones((SEQ, 1), jnp.float32)
    k_ref[...] = kk
    v1_ref[...] = jnp.concatenate([vv, ones], axis=1)
    kv1_ref[...] = jnp.concatenate([kk, vv, ones], axis=1)


def _head(qh, k, v1, kv1, gate):
    s = _dot_t(qh, k)
    m = jnp.max(s, axis=1, keepdims=True)
    rkv = jnp.zeros((SEQ, 2 * D_HEAD + 1), jnp.float32)
    pv = jnp.zeros((SEQ, D_HEAD + 1), jnp.float32)
    for c in range(_NCH):
        sc = s[:, c * _CH:(c + 1) * _CH]
        ohc = (sc >= m).astype(jnp.float32)
        pc = jnp.exp(sc * _SCALE)
        rkv = rkv + _dot(ohc, kv1[c * _CH:(c + 1) * _CH, :])
        pv = pv + _dot(pc, v1[c * _CH:(c + 1) * _CH, :])
    local_out = pv[:, :D_HEAD] / pv[:, D_HEAD:]
    pr = jnp.zeros((SEQ, D_HEAD + 1), jnp.float32)
    for c in range(_NCH):
        rkvc = rkv[c * _CH:(c + 1) * _CH, :]
        s2c = _dot_t(qh, rkvc[:, :D_HEAD])
        p2c = jnp.exp(s2c * _SCALE)
        pr = pr + _dot(p2c, rkvc[:, D_HEAD:])
    r_out = pr[:, :D_HEAD] / pr[:, D_HEAD:]
    return r_out * gate + local_out * (1.0 - gate)


def _main_kernel(q_ref, k_ref, v1_ref, kv1_ref, wq_ref, wct_ref, bias_ref,
                 out_ref):
    t = pl.program_id(0)
    gate = jax.nn.sigmoid(bias_ref[...])
    k = k_ref[...]
    v1 = v1_ref[...]
    kv1 = kv1_ref[...]

    qh2 = _dot_t(q_ref[...], wq_ref[...])
    out_a = _head(qh2[:, :D_HEAD], k, v1, kv1, gate)
    out_b = _head(qh2[:, D_HEAD:], k, v1, kv1, gate)
    contrib = _dot(jnp.concatenate([out_a, out_b], axis=1), wct_ref[...])

    @pl.when(t == 0)
    def _init():
        out_ref[...] = contrib

    @pl.when(t != 0)
    def _acc():
        out_ref[...] += contrib


@functools.partial(jax.jit, static_argnames=())
def kernel(q, kv, w_q, w_kv, w_concat, bias):
    b, l, dm = q.shape
    q2 = q.reshape(l, dm)
    kv2 = kv.reshape(l, dm)
    wct = w_concat.T
    bias2 = bias.reshape(1, D_HEAD)

    k_n, v1_n, kv1_n = pl.pallas_call(
        _proj_kernel,
        out_shape=[
            jax.ShapeDtypeStruct((l, D_HEAD), jnp.float32),
            jax.ShapeDtypeStruct((l, D_HEAD + 1), jnp.float32),
            jax.ShapeDtypeStruct((l, 2 * D_HEAD + 1), jnp.float32),
        ],
    )(kv2, w_kv)

    out = pl.pallas_call(
        _main_kernel,
        grid=(N_HEAD // 2,),
        in_specs=[
            pl.BlockSpec((l, dm), lambda t: (0, 0)),
            pl.BlockSpec((l, D_HEAD), lambda t: (0, 0)),
            pl.BlockSpec((l, D_HEAD + 1), lambda t: (0, 0)),
            pl.BlockSpec((l, 2 * D_HEAD + 1), lambda t: (0, 0)),
            pl.BlockSpec((2 * D_HEAD, dm), lambda t: (t, 0)),
            pl.BlockSpec((2 * D_HEAD, dm), lambda t: (t, 0)),
            pl.BlockSpec((1, D_HEAD), lambda t: (0, 0)),
        ],
        out_specs=pl.BlockSpec((l, dm), lambda t: (0, 0)),
        out_shape=jax.ShapeDtypeStruct((l, dm), jnp.float32),
        compiler_params=pltpu.CompilerParams(
            dimension_semantics=("arbitrary",),
        ),
    )(q2, k_n, v1_n, kv1_n, w_q, wct, bias2)
    return out.reshape(b, l, dm)
